# baseline (speedup 1.0000x reference)
import functools

import jax
import jax.numpy as jnp
from jax import lax
from jax.experimental import pallas as pl
from jax.experimental.pallas import tpu as pltpu
from jax.experimental.pallas import tpu_sc as plsc


def _make_sc_gather(n_ids, emb, dtype):
    info = plsc.get_sparse_core_info()
    nc, ns = info.num_cores, info.num_subcores
    nw = nc * ns
    assert n_ids % (8 * nw) == 0
    b_per_w = n_ids // nw
    chunk = min(128, b_per_w)
    nch = b_per_w // chunk
    mesh = plsc.VectorSubcoreMesh(core_axis_name="c", subcore_axis_name="s")

    @functools.partial(
        pl.kernel,
        mesh=mesh,
        out_type=jax.ShapeDtypeStruct((n_ids, emb), dtype),
        scratch_types=[
            pltpu.VMEM((b_per_w,), jnp.int32),
            pltpu.VMEM((b_per_w, emb), dtype),
            pltpu.SemaphoreType.DMA,
        ],
    )
    def gather(idx_hbm, table_hbm, out_hbm, idx_v, rows_v, sem):
        wid = lax.axis_index("s") * nc + lax.axis_index("c")
        base = wid * b_per_w
        pltpu.sync_copy(idx_hbm.at[pl.ds(base, b_per_w)], idx_v)
        copies = [
            pltpu.async_copy(
                table_hbm.at[idx_v.at[pl.ds(j * chunk, chunk)]],
                rows_v.at[pl.ds(j * chunk, chunk)],
                sem,
            )
            for j in range(nch)
        ]
        for c in copies:
            c.wait()
        pltpu.sync_copy(rows_v, out_hbm.at[pl.ds(base, b_per_w)])

    return gather


def _tc_body(g_ref, pos_ref, tok_ref, wt_ref, b_ref, lnw_ref, lnb_ref, out_ref,
             c_ref):
    @pl.when(pl.program_id(1) == 0)
    def _():
        c_ref[...] = (
            lnw_ref[...] * (pos_ref[...] + tok_ref[0:1, :] + b_ref[...])
            + lnb_ref[...]
        )

    g = g_ref[0].astype(jnp.bfloat16)
    s = g.shape[0]
    zero = jnp.zeros((), jnp.bfloat16)
    rows = lax.broadcasted_iota(jnp.int32, g.shape, 0)
    g_next = jnp.where(rows < s - 1, jnp.roll(g, -1, axis=0), zero)
    g_prev = jnp.where(rows > 0, jnp.roll(g, 1, axis=0), zero)
    trig = jnp.concatenate([g_next, g, g_prev], axis=1)
    x = lax.dot_general(
        trig,
        wt_ref[...],
        (((1,), (0,)), ((), ())),
        preferred_element_type=jnp.float32,
    )
    out_ref[0] = x + c_ref[...]


def kernel(input_ids, word_emb, pos_emb, tok_type_emb, W, b, ln_weight, ln_bias):
    bsz, seq = input_ids.shape
    vocab, emb = word_emb.shape
    hid = W.shape[0]
    n_ids = bsz * seq
    nh = 2
    hid2 = hid // nh

    ids_flat = input_ids.reshape(n_ids).astype(jnp.int32)
    g = _make_sc_gather(n_ids, emb, word_emb.dtype)(ids_flat, word_emb)
    g = g.reshape(bsz, seq, emb)
    wt = (W.T * ln_weight[None, :]).astype(jnp.bfloat16)

    out = pl.pallas_call(
        _tc_body,
        grid=(nh, bsz),
        in_specs=[
            pl.BlockSpec((1, seq, emb), lambda h, i: (i, 0, 0)),
            pl.BlockSpec((seq, hid2), lambda h, i: (0, h)),
            pl.BlockSpec((2, hid2), lambda h, i: (0, h)),
            pl.BlockSpec((3 * emb, hid2), lambda h, i: (0, h)),
            pl.BlockSpec((1, hid2), lambda h, i: (0, h)),
            pl.BlockSpec((1, hid2), lambda h, i: (0, h)),
            pl.BlockSpec((1, hid2), lambda h, i: (0, h)),
        ],
        out_specs=pl.BlockSpec((1, seq, hid2), lambda h, i: (i, 0, h)),
        out_shape=jax.ShapeDtypeStruct((bsz, seq, hid), jnp.float32),
        scratch_shapes=[pltpu.VMEM((seq, hid2), jnp.float32)],
    )(
        g,
        pos_emb[:seq],
        tok_type_emb,
        wt,
        b.reshape(1, hid),
        ln_weight.reshape(1, hid),
        ln_bias.reshape(1, hid),
    )
    return out


# speedup vs baseline: 1.0870x; 1.0870x over previous
import functools

import jax
import jax.numpy as jnp
from jax import lax
from jax.experimental import pallas as pl
from jax.experimental.pallas import tpu as pltpu
from jax.experimental.pallas import tpu_sc as plsc


def _make_sc_gather(n_ids, emb, dtype):
    info = plsc.get_sparse_core_info()
    nc, ns = info.num_cores, info.num_subcores
    nw = nc * ns
    assert n_ids % (8 * nw) == 0
    b_per_w = n_ids // nw
    chunk = min(128, b_per_w)
    nch = b_per_w // chunk
    mesh = plsc.VectorSubcoreMesh(core_axis_name="c", subcore_axis_name="s")

    @functools.partial(
        pl.kernel,
        mesh=mesh,
        out_type=jax.ShapeDtypeStruct((n_ids, emb), dtype),
        scratch_types=[
            pltpu.VMEM((b_per_w,), jnp.int32),
            pltpu.VMEM((b_per_w, emb), dtype),
            pltpu.SemaphoreType.DMA,
        ],
    )
    def gather(idx_hbm, table_hbm, out_hbm, idx_v, rows_v, sem):
        wid = lax.axis_index("s") * nc + lax.axis_index("c")
        base = wid * b_per_w
        pltpu.sync_copy(idx_hbm.at[pl.ds(base, b_per_w)], idx_v)
        copies = [
            pltpu.async_copy(
                table_hbm.at[idx_v.at[pl.ds(j * chunk, chunk)]],
                rows_v.at[pl.ds(j * chunk, chunk)],
                sem,
            )
            for j in range(nch)
        ]
        for c in copies:
            c.wait()
        pltpu.sync_copy(rows_v, out_hbm.at[pl.ds(base, b_per_w)])

    return gather


def _tc_body(g_ref, pos_ref, tok_ref, wt_ref, b_ref, lnw_ref, lnb_ref, out_ref,
             c_ref):
    @pl.when(pl.program_id(0) == 0)
    def _():
        c_ref[...] = (
            lnw_ref[...] * (pos_ref[...] + tok_ref[0:1, :] + b_ref[...])
            + lnb_ref[...]
        )

    g = g_ref[0].astype(jnp.bfloat16)
    s = g.shape[0]
    zero = jnp.zeros((), jnp.bfloat16)
    rows = lax.broadcasted_iota(jnp.int32, g.shape, 0)
    g_next = jnp.where(rows < s - 1, jnp.roll(g, -1, axis=0), zero)
    g_prev = jnp.where(rows > 0, jnp.roll(g, 1, axis=0), zero)
    trig = jnp.concatenate([g_next, g, g_prev], axis=1)
    x = lax.dot_general(
        trig,
        wt_ref[...],
        (((1,), (0,)), ((), ())),
        preferred_element_type=jnp.float32,
    )
    out_ref[0] = x + c_ref[...]


def kernel(input_ids, word_emb, pos_emb, tok_type_emb, W, b, ln_weight, ln_bias):
    bsz, seq = input_ids.shape
    vocab, emb = word_emb.shape
    hid = W.shape[0]
    n_ids = bsz * seq

    ids_flat = input_ids.reshape(n_ids).astype(jnp.int32)
    g = _make_sc_gather(n_ids, emb, word_emb.dtype)(ids_flat, word_emb)
    g = g.reshape(bsz, seq, emb)
    wt = (W.T * ln_weight[None, :]).astype(jnp.bfloat16)

    out = pl.pallas_call(
        _tc_body,
        grid=(bsz,),
        in_specs=[
            pl.BlockSpec((1, seq, emb), lambda i: (i, 0, 0)),
            pl.BlockSpec((seq, hid), lambda i: (0, 0)),
            pl.BlockSpec((2, hid), lambda i: (0, 0)),
            pl.BlockSpec((3 * emb, hid), lambda i: (0, 0)),
            pl.BlockSpec((1, hid), lambda i: (0, 0)),
            pl.BlockSpec((1, hid), lambda i: (0, 0)),
            pl.BlockSpec((1, hid), lambda i: (0, 0)),
        ],
        out_specs=pl.BlockSpec((1, seq, hid), lambda i: (i, 0, 0)),
        out_shape=jax.ShapeDtypeStruct((bsz, seq, hid), jnp.float32),
        scratch_shapes=[pltpu.VMEM((seq, hid), jnp.float32)],
    )(
        g,
        pos_emb[:seq],
        tok_type_emb,
        wt,
        b.reshape(1, hid),
        ln_weight.reshape(1, hid),
        ln_bias.reshape(1, hid),
    )
    return out


# speedup vs baseline: 1.0872x; 1.0002x over previous
import functools

import jax
import jax.numpy as jnp
from jax import lax
from jax.experimental import pallas as pl
from jax.experimental.pallas import tpu as pltpu
from jax.experimental.pallas import tpu_sc as plsc


def _make_sc_gather(n_ids, emb, dtype):
    info = plsc.get_sparse_core_info()
    nc, ns = info.num_cores, info.num_subcores
    nw = nc * ns
    assert n_ids % (8 * nw) == 0
    b_per_w = n_ids // nw
    chunk = min(128, b_per_w)
    nch = b_per_w // chunk
    mesh = plsc.VectorSubcoreMesh(core_axis_name="c", subcore_axis_name="s")

    @functools.partial(
        pl.kernel,
        mesh=mesh,
        out_type=jax.ShapeDtypeStruct((n_ids, emb), dtype),
        scratch_types=[
            pltpu.VMEM((b_per_w,), jnp.int32),
            pltpu.VMEM((b_per_w, emb), dtype),
            pltpu.SemaphoreType.DMA,
        ],
    )
    def gather(idx_hbm, table_hbm, out_hbm, idx_v, rows_v, sem):
        wid = lax.axis_index("s") * nc + lax.axis_index("c")
        base = wid * b_per_w
        pltpu.sync_copy(idx_hbm.at[pl.ds(base, b_per_w)], idx_v)
        copies = [
            pltpu.async_copy(
                table_hbm.at[idx_v.at[pl.ds(j * chunk, chunk)]],
                rows_v.at[pl.ds(j * chunk, chunk)],
                sem,
            )
            for j in range(nch)
        ]
        for c in copies:
            c.wait()
        pltpu.sync_copy(rows_v, out_hbm.at[pl.ds(base, b_per_w)])

    return gather


def _tc_body(g_ref, pos_ref, tok_ref, wt_ref, b_ref, lnw_ref, lnb_ref, out_ref,
             c_ref):
    @pl.when(pl.program_id(0) == 0)
    def _():
        c_ref[...] = (
            lnw_ref[...] * (pos_ref[...] + tok_ref[0:1, :] + b_ref[...])
            + lnb_ref[...]
        )

    g = g_ref[0].astype(jnp.bfloat16)
    s = g.shape[0]
    zero = jnp.zeros((), jnp.bfloat16)
    rows = lax.broadcasted_iota(jnp.int32, g.shape, 0)
    g_next = jnp.where(rows < s - 1, jnp.roll(g, -1, axis=0), zero)
    g_prev = jnp.where(rows > 0, jnp.roll(g, 1, axis=0), zero)
    trig = jnp.concatenate([g_next, g, g_prev], axis=1)
    x = lax.dot_general(
        trig,
        wt_ref[...],
        (((1,), (0,)), ((), ())),
        preferred_element_type=jnp.float32,
    )
    out_ref[0] = x + c_ref[...]


def kernel(input_ids, word_emb, pos_emb, tok_type_emb, W, b, ln_weight, ln_bias):
    bsz, seq = input_ids.shape
    vocab, emb = word_emb.shape
    hid = W.shape[0]
    n_ids = bsz * seq

    ids_flat = input_ids.reshape(n_ids).astype(jnp.int32)
    g = _make_sc_gather(n_ids, emb, word_emb.dtype)(ids_flat, word_emb)
    g = g.reshape(bsz, seq, emb)
    wt = (W.T * ln_weight[None, :]).astype(jnp.bfloat16)

    out = pl.pallas_call(
        _tc_body,
        grid=(bsz,),
        in_specs=[
            pl.BlockSpec((1, seq, emb), lambda i: (i, 0, 0)),
            pl.BlockSpec((seq, hid), lambda i: (0, 0)),
            pl.BlockSpec((2, hid), lambda i: (0, 0)),
            pl.BlockSpec((3 * emb, hid), lambda i: (0, 0)),
            pl.BlockSpec((1, hid), lambda i: (0, 0)),
            pl.BlockSpec((1, hid), lambda i: (0, 0)),
            pl.BlockSpec((1, hid), lambda i: (0, 0)),
        ],
        out_specs=pl.BlockSpec((1, seq, hid), lambda i: (i, 0, 0)),
        out_shape=jax.ShapeDtypeStruct((bsz, seq, hid), jnp.float32),
        scratch_shapes=[pltpu.VMEM((seq, hid), jnp.float32)],
        compiler_params=pltpu.CompilerParams(
            dimension_semantics=("parallel",),
        ),
    )(
        g,
        pos_emb[:seq],
        tok_type_emb,
        wt,
        b.reshape(1, hid),
        ln_weight.reshape(1, hid),
        ln_bias.reshape(1, hid),
    )
    return out
